# baseline (speedup 1.0000x reference)
import functools

import jax
import jax.numpy as jnp
from jax import lax
from jax.experimental import pallas as pl
from jax.experimental.pallas import tpu as pltpu
from jax.experimental.pallas import tpu_sc as plsc

_B, _T, _D = 4, 4096, 2048
_T_SC = 2048
_T_TC = _T - _T_SC
_TBLK = 2048
_G = 16
_NWORK = 32
_WPB = _NWORK // _B
_NW = _T_SC // _WPB
_NGRP = _NW // _G


def _tc_body(h_ref, w_ref, out_ref, carry_ref):
    t = pl.program_id(1)
    h = h_ref[0]
    d = h.shape[-1]
    w1 = w_ref[:, :d]
    w2 = w_ref[:, d:]
    av = jnp.sum(h * w1, axis=1)[None, :]
    cv = jnp.sum(h * w2, axis=1)[None, :]
    prev = jnp.where(t == 0, 0.0, carry_ref[0])
    shifted = jnp.roll(cv, 1, axis=1)
    col = lax.broadcasted_iota(jnp.int32, cv.shape, 1)
    out_ref[0] = av + jnp.where(col == 0, prev, shifted)
    carry_ref[0] = cv[0, cv.shape[1] - 1]


def _tc_part(hidden_states, W):
    b, t, d = hidden_states.shape
    nt = _T_TC // _TBLK
    out = pl.pallas_call(
        _tc_body,
        grid=(b, nt),
        in_specs=[
            pl.BlockSpec((1, _TBLK, d), lambda i, j: (i, j, 0)),
            pl.BlockSpec((1, 2 * d), lambda i, j: (0, 0)),
        ],
        out_specs=pl.BlockSpec((1, 1, _TBLK), lambda i, j: (i * nt + j, 0, 0)),
        out_shape=jax.ShapeDtypeStruct((b * nt, 1, _TBLK), hidden_states.dtype),
        scratch_shapes=[pltpu.SMEM((1,), jnp.float32)],
        compiler_params=pltpu.CompilerParams(
            dimension_semantics=("parallel", "arbitrary"),
        ),
    )(hidden_states, W)
    return out.reshape(b, _T_TC)


def _reduce_body(p_ref, out_ref):
    out_ref[...] = jnp.sum(p_ref[...], axis=1, keepdims=True)


def _reduce_part(partials):
    n = partials.shape[0]
    out = pl.pallas_call(
        _reduce_body,
        out_shape=jax.ShapeDtypeStruct((n, 1), jnp.float32),
    )(partials)
    return out.reshape(_B, _T_SC)


def _sc_compute_group(buf, w_v, o_v, o_base):
    nacc = _G + 1

    def kbody(kb, carry):
        a1s, a2s = carry
        koff = kb * 16
        w1 = w_v[pl.ds(koff, 16)]
        w2 = w_v[pl.ds(_D + koff, 16)]
        n1 = []
        n2 = []
        for r in range(nacc):
            h = buf[pl.ds(r * _D + koff, 16)]
            n1.append(a1s[r] + h * w1)
            n2.append(a2s[r] + h * w2)
        return tuple(n1), tuple(n2)

    zero = jnp.zeros((16,), jnp.float32)
    a1s, a2s = lax.fori_loop(
        0, _D // 16, kbody,
        (tuple(zero for _ in range(nacc)), tuple(zero for _ in range(nacc))),
    )
    for j in range(_G):
        o_v[pl.ds((o_base + j) * 16, 16)] = a1s[j + 1] + a2s[j]


def _sc_body(h_hbm, w_hbm, out_hbm, w_v, buf0, buf1, o_v, sem0, sem1):
    c = lax.axis_index("c")
    s = lax.axis_index("s")
    wid = s * 2 + c
    bidx = wid // _WPB
    chunk = wid % _WPB
    ts = _T_TC + chunk * _NW
    row0 = bidx * _T + ts - 1

    pltpu.sync_copy(w_hbm, w_v)
    bufs = (buf0, buf1)
    sems = (sem0, sem1)
    copies = [None] * _NGRP
    copies[0] = pltpu.async_copy(
        h_hbm.at[pl.ds(row0 * _D, (_G + 1) * _D)], buf0, sem0)
    for g in range(_NGRP):
        if g + 1 < _NGRP:
            copies[g + 1] = pltpu.async_copy(
                h_hbm.at[pl.ds((row0 + (g + 1) * _G) * _D, (_G + 1) * _D)],
                bufs[(g + 1) % 2], sems[(g + 1) % 2])
        copies[g].wait()
        _sc_compute_group(bufs[g % 2], w_v, o_v, g * _G)
    pltpu.sync_copy(
        o_v, out_hbm.at[pl.ds((bidx * _T_SC + chunk * _NW) * 16, _NW * 16)])


@functools.partial(
    pl.kernel,
    out_type=jax.ShapeDtypeStruct((_B * _T_SC * 16,), jnp.float32),
    mesh=plsc.VectorSubcoreMesh(core_axis_name="c", subcore_axis_name="s"),
    scratch_types=[
        pltpu.VMEM((2 * _D,), jnp.float32),
        pltpu.VMEM(((_G + 1) * _D,), jnp.float32),
        pltpu.VMEM(((_G + 1) * _D,), jnp.float32),
        pltpu.VMEM((_NW * 16,), jnp.float32),
        pltpu.SemaphoreType.DMA,
        pltpu.SemaphoreType.DMA,
    ],
)
def _sc_part(h_hbm, w_hbm, out_hbm, w_v, buf0, buf1, o_v, sem0, sem1):
    _sc_body(h_hbm, w_hbm, out_hbm, w_v, buf0, buf1, o_v, sem0, sem1)


@jax.jit
def kernel(hidden_states, W):
    b, t, d = hidden_states.shape
    tc = _tc_part(hidden_states, W)
    scp = _sc_part(hidden_states.reshape(-1), W.reshape(-1))
    sc = _reduce_part(scp.reshape(_B * _T_SC, 16))
    return jnp.concatenate([tc, sc], axis=1)


# speedup vs baseline: 2.2373x; 2.2373x over previous
import functools

import jax
import jax.numpy as jnp
from jax import lax
from jax.experimental import pallas as pl
from jax.experimental.pallas import tpu as pltpu
from jax.experimental.pallas import tpu_sc as plsc

_B, _T, _D = 4, 4096, 2048
_T_SC = 2048
_T_TC = _T - _T_SC
_TBLK = 2048
_G = 16
_NWORK = 32
_WPB = _NWORK // _B
_NW = _T_SC // _WPB
_NGRP = _NW // _G


def _tc_body(h_ref, w_ref, out_ref, carry_ref):
    t = pl.program_id(1)
    h = h_ref[0]
    d = h.shape[-1]
    w1 = w_ref[:, :d]
    w2 = w_ref[:, d:]
    av = jnp.sum(h * w1, axis=1)[None, :]
    cv = jnp.sum(h * w2, axis=1)[None, :]
    prev = jnp.where(t == 0, 0.0, carry_ref[0])
    shifted = jnp.roll(cv, 1, axis=1)
    col = lax.broadcasted_iota(jnp.int32, cv.shape, 1)
    out_ref[0] = av + jnp.where(col == 0, prev, shifted)
    carry_ref[0] = cv[0, cv.shape[1] - 1]


def _tc_part(hidden_states, W):
    b, t, d = hidden_states.shape
    nt = _T_TC // _TBLK
    out = pl.pallas_call(
        _tc_body,
        grid=(b, nt),
        in_specs=[
            pl.BlockSpec((1, _TBLK, d), lambda i, j: (i, j, 0)),
            pl.BlockSpec((1, 2 * d), lambda i, j: (0, 0)),
        ],
        out_specs=pl.BlockSpec((1, 1, _TBLK), lambda i, j: (i * nt + j, 0, 0)),
        out_shape=jax.ShapeDtypeStruct((b * nt, 1, _TBLK), hidden_states.dtype),
        scratch_shapes=[pltpu.SMEM((1,), jnp.float32)],
        compiler_params=pltpu.CompilerParams(
            dimension_semantics=("parallel", "arbitrary"),
        ),
    )(hidden_states, W)
    return out.reshape(b, _T_TC)


def _reduce_body(p_ref, out_ref):
    out_ref[...] = jnp.sum(p_ref[...], axis=1, keepdims=True)


def _reduce_part(partials):
    n = partials.shape[0]
    out = pl.pallas_call(
        _reduce_body,
        out_shape=jax.ShapeDtypeStruct((n, 1), jnp.float32),
    )(partials)
    return out.reshape(_B, _T_SC)


def _sc_dot_rows(buf, w_v, nrows, row0=0):
    def kbody(kb, carry):
        a1s, a2s = carry
        koff = kb * 16
        w1 = w_v[0, pl.ds(koff, 16)]
        w2 = w_v[0, pl.ds(_D + koff, 16)]
        n1 = []
        n2 = []
        for r in range(nrows):
            h = buf[row0 + r, pl.ds(koff, 16)]
            n1.append(a1s[r] + h * w1)
            n2.append(a2s[r] + h * w2)
        return tuple(n1), tuple(n2)

    zero = jnp.zeros((16,), jnp.float32)
    return lax.fori_loop(
        0, _D // 16, kbody,
        (tuple(zero for _ in range(nrows)), tuple(zero for _ in range(nrows))),
    )


def _sc_body(h_hbm, w_hbm, out_hbm, w_v, pbuf, buf0, buf1, o_v, semp, sem0, sem1):
    c = lax.axis_index("c")
    s = lax.axis_index("s")
    wid = s * 2 + c
    bidx = wid // _WPB
    chunk = wid % _WPB
    ts = _T_TC + chunk * _NW

    cpp = pltpu.async_copy(h_hbm.at[bidx, pl.ds(ts - 8, 8)], pbuf, semp)
    bufs = (buf0, buf1)
    sems = (sem0, sem1)
    copies = [None] * _NGRP
    copies[0] = pltpu.async_copy(h_hbm.at[bidx, pl.ds(ts, _G)], buf0, sem0)
    pltpu.sync_copy(w_hbm, w_v)

    cpp.wait()
    _, p2 = _sc_dot_rows(pbuf, w_v, 1, row0=7)
    carry = p2[0]

    for g in range(_NGRP):
        if g + 1 < _NGRP:
            copies[g + 1] = pltpu.async_copy(
                h_hbm.at[bidx, pl.ds(ts + (g + 1) * _G, _G)],
                bufs[(g + 1) % 2], sems[(g + 1) % 2])
        copies[g].wait()
        a1s, a2s = _sc_dot_rows(bufs[g % 2], w_v, _G)
        for j in range(_G):
            prev = carry if j == 0 else a2s[j - 1]
            o_v[pl.ds((g * _G + j) * 16, 16)] = a1s[j] + prev
        carry = a2s[_G - 1]
    pltpu.sync_copy(
        o_v, out_hbm.at[pl.ds((bidx * _T_SC + chunk * _NW) * 16, _NW * 16)])


@functools.partial(
    pl.kernel,
    out_type=jax.ShapeDtypeStruct((_B * _T_SC * 16,), jnp.float32),
    mesh=plsc.VectorSubcoreMesh(core_axis_name="c", subcore_axis_name="s"),
    scratch_types=[
        pltpu.VMEM((1, 2 * _D), jnp.float32),
        pltpu.VMEM((8, _D), jnp.float32),
        pltpu.VMEM((_G, _D), jnp.float32),
        pltpu.VMEM((_G, _D), jnp.float32),
        pltpu.VMEM((_NW * 16,), jnp.float32),
        pltpu.SemaphoreType.DMA,
        pltpu.SemaphoreType.DMA,
        pltpu.SemaphoreType.DMA,
    ],
)
def _sc_part(h_hbm, w_hbm, out_hbm, w_v, pbuf, buf0, buf1, o_v, semp, sem0, sem1):
    _sc_body(h_hbm, w_hbm, out_hbm, w_v, pbuf, buf0, buf1, o_v, semp, sem0, sem1)


@jax.jit
def kernel(hidden_states, W):
    b, t, d = hidden_states.shape
    tc = _tc_part(hidden_states, W)
    scp = _sc_part(hidden_states, W)
    sc = _reduce_part(scp.reshape(_B * _T_SC, 16))
    return jnp.concatenate([tc, sc], axis=1)


# speedup vs baseline: 2.7021x; 1.2078x over previous
import functools

import jax
import jax.numpy as jnp
from jax import lax
from jax.experimental import pallas as pl
from jax.experimental.pallas import tpu as pltpu
from jax.experimental.pallas import tpu_sc as plsc

_B, _T, _D = 4, 4096, 2048
_T_SC = 1024
_T_TC = _T - _T_SC
_TBLK = 1536
_G = 16
_NWORK = 32
_WPB = _NWORK // _B
_NW = _T_SC // _WPB
_NGRP = _NW // _G


def _tc_body(h_ref, w_ref, out_ref, carry_ref):
    t = pl.program_id(1)
    h = h_ref[0]
    d = h.shape[-1]
    w1 = w_ref[:, :d]
    w2 = w_ref[:, d:]
    av = jnp.sum(h * w1, axis=1)[None, :]
    cv = jnp.sum(h * w2, axis=1)[None, :]
    prev = jnp.where(t == 0, 0.0, carry_ref[0])
    shifted = jnp.roll(cv, 1, axis=1)
    col = lax.broadcasted_iota(jnp.int32, cv.shape, 1)
    out_ref[0] = av + jnp.where(col == 0, prev, shifted)
    carry_ref[0] = cv[0, cv.shape[1] - 1]


def _tc_part(hidden_states, W):
    b, t, d = hidden_states.shape
    nt = _T_TC // _TBLK
    out = pl.pallas_call(
        _tc_body,
        grid=(b, nt),
        in_specs=[
            pl.BlockSpec((1, _TBLK, d), lambda i, j: (i, j, 0)),
            pl.BlockSpec((1, 2 * d), lambda i, j: (0, 0)),
        ],
        out_specs=pl.BlockSpec((1, 1, _TBLK), lambda i, j: (i * nt + j, 0, 0)),
        out_shape=jax.ShapeDtypeStruct((b * nt, 1, _TBLK), hidden_states.dtype),
        scratch_shapes=[pltpu.SMEM((1,), jnp.float32)],
        compiler_params=pltpu.CompilerParams(
            dimension_semantics=("parallel", "arbitrary"),
        ),
    )(hidden_states, W)
    return out.reshape(b, _T_TC)


def _sc_dot_rows(buf, w_v, nrows, row0=0):
    def kbody(kb, carry):
        a1s, a2s = carry
        koff = kb * 16
        w1 = w_v[0, pl.ds(koff, 16)]
        w2 = w_v[0, pl.ds(_D + koff, 16)]
        n1 = []
        n2 = []
        for r in range(nrows):
            h = buf[row0 + r, pl.ds(koff, 16)]
            n1.append(a1s[r] + h * w1)
            n2.append(a2s[r] + h * w2)
        return tuple(n1), tuple(n2)

    zero = jnp.zeros((16,), jnp.float32)
    return lax.fori_loop(
        0, _D // 16, kbody,
        (tuple(zero for _ in range(nrows)), tuple(zero for _ in range(nrows))),
    )


def _sc_reduce_group(v, m_v, o_v, o_base):
    for sh in (8, 4, 2, 1):
        for j in range(_G):
            m_v[pl.ds(j * 32, 16)] = v[j]
        for j in range(_G):
            v[j] = v[j] + m_v[pl.ds(j * 32 + sh, 16)]
    for j in range(_G):
        o_v[pl.ds(o_base + j, 16)] = v[j]


def _sc_body(h_hbm, w_hbm, out_hbm, w_v, pbuf, buf0, buf1, m_v, o_v,
             semp, sem0, sem1):
    c = lax.axis_index("c")
    s = lax.axis_index("s")
    wid = s * 2 + c
    bidx = wid // _WPB
    chunk = wid % _WPB
    ts = _T_TC + chunk * _NW

    cpp = pltpu.async_copy(h_hbm.at[bidx, pl.ds(ts - 8, 8)], pbuf, semp)
    bufs = (buf0, buf1)
    sems = (sem0, sem1)
    copies = [None] * _NGRP
    copies[0] = pltpu.async_copy(h_hbm.at[bidx, pl.ds(ts, _G)], buf0, sem0)
    pltpu.sync_copy(w_hbm, w_v)
    zero = jnp.zeros((16,), jnp.float32)
    for j in range(_G):
        m_v[pl.ds(j * 32 + 16, 16)] = zero

    cpp.wait()
    _, p2 = _sc_dot_rows(pbuf, w_v, 1, row0=7)
    carry = p2[0]

    for g in range(_NGRP):
        if g + 1 < _NGRP:
            copies[g + 1] = pltpu.async_copy(
                h_hbm.at[bidx, pl.ds(ts + (g + 1) * _G, _G)],
                bufs[(g + 1) % 2], sems[(g + 1) % 2])
        copies[g].wait()
        a1s, a2s = _sc_dot_rows(bufs[g % 2], w_v, _G)
        v = [a1s[j] + (carry if j == 0 else a2s[j - 1]) for j in range(_G)]
        carry = a2s[_G - 1]
        _sc_reduce_group(v, m_v, o_v, g * _G)
    pltpu.sync_copy(o_v.at[pl.ds(0, _NW)],
                    out_hbm.at[bidx, pl.ds(chunk * _NW, _NW)])


@functools.partial(
    pl.kernel,
    out_type=jax.ShapeDtypeStruct((_B, _T_SC), jnp.float32),
    mesh=plsc.VectorSubcoreMesh(core_axis_name="c", subcore_axis_name="s"),
    scratch_types=[
        pltpu.VMEM((1, 2 * _D), jnp.float32),
        pltpu.VMEM((8, _D), jnp.float32),
        pltpu.VMEM((_G, _D), jnp.float32),
        pltpu.VMEM((_G, _D), jnp.float32),
        pltpu.VMEM((_G * 32,), jnp.float32),
        pltpu.VMEM((_NW + 16,), jnp.float32),
        pltpu.SemaphoreType.DMA,
        pltpu.SemaphoreType.DMA,
        pltpu.SemaphoreType.DMA,
    ],
)
def _sc_part(h_hbm, w_hbm, out_hbm, w_v, pbuf, buf0, buf1, m_v, o_v,
             semp, sem0, sem1):
    _sc_body(h_hbm, w_hbm, out_hbm, w_v, pbuf, buf0, buf1, m_v, o_v,
             semp, sem0, sem1)


@jax.jit
def kernel(hidden_states, W):
    b, t, d = hidden_states.shape
    tc = _tc_part(hidden_states, W)
    sc = _sc_part(hidden_states, W)
    return jnp.concatenate([tc, sc], axis=1)


# speedup vs baseline: 3.8243x; 1.4153x over previous
import jax
import jax.numpy as jnp
from jax import lax
from jax.experimental import pallas as pl
from jax.experimental.pallas import tpu as pltpu

_TBLK = 2048


def _body(h1_ref, h2_ref, w_ref, out_ref, carry_ref):
    t = pl.program_id(1)
    h1 = h1_ref[0]
    h2 = h2_ref[0]
    hd = h1.shape[-1]
    d = 2 * hd
    w1a = w_ref[:, :hd]
    w1b = w_ref[:, hd:d]
    w2a = w_ref[:, d:d + hd]
    w2b = w_ref[:, d + hd:]
    av = (jnp.sum(h1 * w1a, axis=1) + jnp.sum(h2 * w1b, axis=1))[None, :]
    cv = (jnp.sum(h1 * w2a, axis=1) + jnp.sum(h2 * w2b, axis=1))[None, :]
    prev = jnp.where(t == 0, 0.0, carry_ref[0])
    shifted = jnp.roll(cv, 1, axis=1)
    col = lax.broadcasted_iota(jnp.int32, cv.shape, 1)
    out_ref[0] = av + jnp.where(col == 0, prev, shifted)
    carry_ref[0] = cv[0, cv.shape[1] - 1]


@jax.jit
def kernel(hidden_states, W):
    b, t, d = hidden_states.shape
    nt = t // _TBLK
    out = pl.pallas_call(
        _body,
        grid=(b, nt),
        in_specs=[
            pl.BlockSpec((1, _TBLK, d // 2), lambda i, j: (i, j, 0)),
            pl.BlockSpec((1, _TBLK, d // 2), lambda i, j: (i, j, 1)),
            pl.BlockSpec((1, 2 * d), lambda i, j: (0, 0)),
        ],
        out_specs=pl.BlockSpec((1, 1, _TBLK), lambda i, j: (i * nt + j, 0, 0)),
        out_shape=jax.ShapeDtypeStruct((b * nt, 1, _TBLK), hidden_states.dtype),
        scratch_shapes=[pltpu.SMEM((1,), jnp.float32)],
        compiler_params=pltpu.CompilerParams(
            dimension_semantics=("parallel", "arbitrary"),
        ),
    )(hidden_states, hidden_states, W)
    return out.reshape(b, t)


# speedup vs baseline: 3.8326x; 1.0022x over previous
import jax
import jax.numpy as jnp
from jax import lax
from jax.experimental import pallas as pl
from jax.experimental.pallas import tpu as pltpu

_TBLK = 2048


def _body(h_ref, w_ref, out_ref, carry_ref):
    t = pl.program_id(1)
    h = h_ref[0]
    d = h.shape[-1]
    w1 = w_ref[:, :d]
    w2 = w_ref[:, d:]
    av = jnp.sum(h * w1, axis=1)[None, :]
    cv = jnp.sum(h * w2, axis=1)[None, :]
    prev = jnp.where(t == 0, 0.0, carry_ref[0])
    shifted = jnp.roll(cv, 1, axis=1)
    col = lax.broadcasted_iota(jnp.int32, cv.shape, 1)
    out_ref[0] = av + jnp.where(col == 0, prev, shifted)
    carry_ref[0] = cv[0, cv.shape[1] - 1]


@jax.jit
def kernel(hidden_states, W):
    b, t, d = hidden_states.shape
    nt = t // _TBLK
    out = pl.pallas_call(
        _body,
        grid=(b, nt),
        in_specs=[
            pl.BlockSpec((1, _TBLK, d), lambda i, j: (i, j, 0)),
            pl.BlockSpec((1, 2 * d), lambda i, j: (0, 0)),
        ],
        out_specs=pl.BlockSpec((1, 1, _TBLK), lambda i, j: (i * nt + j, 0, 0)),
        out_shape=jax.ShapeDtypeStruct((b * nt, 1, _TBLK), hidden_states.dtype),
        scratch_shapes=[pltpu.SMEM((1,), jnp.float32)],
        compiler_params=pltpu.CompilerParams(
            dimension_semantics=("parallel", "arbitrary"),
        ),
    )(hidden_states, W)
    return out.reshape(b, t)
